# baseline (speedup 1.0000x reference)
import functools
import jax
import jax.numpy as jnp
from jax import lax
from jax.experimental import pallas as pl
from jax.experimental.pallas import tpu as pltpu
from jax.experimental.pallas import tpu_sc as plsc

_NC, _NS = 2, 16
_NW = _NC * _NS


def kernel(x, embedding):
    B, S = x.shape
    D = embedding.shape[1]
    rows_w = S // _NW

    mesh = plsc.VectorSubcoreMesh(core_axis_name="c", subcore_axis_name="s")

    @functools.partial(
        pl.kernel,
        out_type=jax.ShapeDtypeStruct((B, S, D), jnp.float32),
        mesh=mesh,
        scratch_types=[pltpu.SemaphoreType.DMA],
    )
    def sc_copy(emb_hbm, out_hbm, sem):
        wid = lax.axis_index("s") * _NC + lax.axis_index("c")
        base = wid * rows_w
        cps = [
            pltpu.make_async_copy(
                emb_hbm.at[pl.ds(base, rows_w)],
                out_hbm.at[b, pl.ds(base, rows_w)], sem)
            for b in range(B)
        ]
        for c in cps:
            c.start()
        for c in cps:
            c.wait()

    return sc_copy(embedding[:S])


# speedup vs baseline: 69.8945x; 69.8945x over previous
import jax
import jax.numpy as jnp
from jax.experimental import pallas as pl


def _body(emb_ref, out_ref):
    out_ref[0] = emb_ref[...]


def kernel(x, embedding):
    B, S = x.shape
    D = embedding.shape[1]
    ROWS = 2048
    out = pl.pallas_call(
        _body,
        grid=(S // ROWS, B),
        in_specs=[pl.BlockSpec((ROWS, D), lambda i, b: (i, 0))],
        out_specs=pl.BlockSpec((1, ROWS, D), lambda i, b: (b, i, 0)),
        out_shape=jax.ShapeDtypeStruct((B, S, D), jnp.float32),
    )(embedding[:S])
    return out
